# baseline (speedup 1.0000x reference)
import functools

import jax
import jax.numpy as jnp
from jax import lax
from jax.experimental import pallas as pl
from jax.experimental.pallas import tpu as pltpu
from jax.experimental.pallas import tpu_sc as plsc

_K = 16
_DILATION = 2
_KD = _K * _DILATION

_L = 16
_NW = 32
_RB = 8
_CAND_CAP = 256
_CAPL = 256


def _dist_kernel(a_rows_ref, a_full_ref, out_ref):
    a_rows = a_rows_ref[0]
    a_full = a_full_ref[0]
    inner = -2.0 * jax.lax.dot_general(
        a_rows, a_full, (((0,), (0,)), ((), ())),
        preferred_element_type=jnp.float32)
    sq_rows = jnp.sum(a_rows * a_rows, axis=0)
    sq_full = jnp.sum(a_full * a_full, axis=0)
    t = (sq_rows[:, None] + inner) + sq_full[None, :]
    out_ref[...] = -t


def _neg_adj(a):
    B, D, N = a.shape
    R = 256
    return pl.pallas_call(
        _dist_kernel,
        grid=(B, N // R),
        in_specs=[
            pl.BlockSpec((1, D, R), lambda b, r: (b, 0, r)),
            pl.BlockSpec((1, D, N), lambda b, r: (b, 0, 0)),
        ],
        out_specs=pl.BlockSpec((R, N), lambda b, r: (b * (N // R) + r, 0)),
        out_shape=jax.ShapeDtypeStruct((B * N, N), jnp.float32),
    )(a, a)


def _merge16(ak, bk):
    rb = lax.rev(bk, (0,))
    u = jnp.maximum(ak, rb)
    l = jnp.minimum(ak, rb)
    us, _ = plsc.sort_key_val(u, u, descending=True)
    ls, _ = plsc.sort_key_val(l, l, descending=True)
    return us, ls


def _make_sc_topk(rows_total, n):
    rows_per_w = rows_total // _NW
    nblk = rows_per_w // _RB
    nv = n // _L
    blk_words = _RB * n
    out_words = rows_per_w * _K

    mesh = plsc.VectorSubcoreMesh(core_axis_name="c", subcore_axis_name="s")

    @functools.partial(
        pl.kernel,
        mesh=mesh,
        out_type=jax.ShapeDtypeStruct((rows_total * _K,), jnp.int32),
        scratch_types=[
            pltpu.VMEM((_RB, n), jnp.float32),
            pltpu.VMEM((_RB, n), jnp.float32),
            pltpu.VMEM((_L * _CAPL,), jnp.int32),
            pltpu.VMEM((out_words + _L,), jnp.int32),
            pltpu.SemaphoreType.DMA,
            pltpu.SemaphoreType.DMA,
        ],
        compiler_params=pltpu.CompilerParams(needs_layout_passes=False),
    )
    def sc_topk(neg_hbm, out_hbm, blk_a, blk_b, cand_v, out_v, sem_a, sem_b):
        cid = lax.axis_index("c")
        sid = lax.axis_index("s")
        wid = sid * 2 + cid
        row_base = wid * rows_per_w

        neginf = jnp.float32(-jnp.inf)
        iota = lax.iota(jnp.int32, _L)
        evenmask = (iota & 1) == 0

        def src(blk):
            return neg_hbm.at[pl.ds(row_base + blk * _RB, _RB)]

        def process_rows(blk_v, blk):
            for r in range(_RB):

                def pa(i, ms):
                    m1, m2, m3, m4 = ms
                    gb = i * (4 * _L)
                    x0 = blk_v[r, pl.ds(gb, _L)]
                    x1 = blk_v[r, pl.ds(gb + _L, _L)]
                    x2 = blk_v[r, pl.ds(gb + 2 * _L, _L)]
                    x3 = blk_v[r, pl.ds(gb + 3 * _L, _L)]
                    a = jnp.maximum(x0, x1)
                    b = jnp.minimum(x0, x1)
                    c = jnp.maximum(x2, x3)
                    d = jnp.minimum(x2, x3)
                    g1 = jnp.maximum(a, c)
                    c2 = jnp.minimum(a, c)
                    b2 = jnp.maximum(b, d)
                    g4 = jnp.minimum(b, d)
                    g2 = jnp.maximum(c2, b2)
                    g3 = jnp.minimum(c2, b2)
                    h1 = jnp.maximum(m1, g4)
                    h2 = jnp.maximum(m2, g3)
                    h3 = jnp.maximum(m3, g2)
                    h4 = jnp.maximum(m4, g1)
                    p1 = jnp.maximum(h1, h3)
                    p3 = jnp.minimum(h1, h3)
                    p2 = jnp.maximum(h2, h4)
                    p4 = jnp.minimum(h2, h4)
                    return (jnp.maximum(p1, p2), jnp.minimum(p1, p2),
                            jnp.maximum(p3, p4), jnp.minimum(p3, p4))

                finf = jnp.full((_L,), neginf)
                m1, m2, m3, m4 = plsc.parallel_loop(
                    0, nv // 4, unroll=4,
                    carry=(finf, finf, finf, finf))(pa)

                s1, _ = plsc.sort_key_val(m1, m1, descending=True)
                s2, _ = plsc.sort_key_val(m2, m2, descending=True)
                s3, _ = plsc.sort_key_val(m3, m3, descending=True)
                s4, _ = plsc.sort_key_val(m4, m4, descending=True)
                a0, a1 = _merge16(s1, s2)
                b0, b1 = _merge16(s3, s4)
                h0 = jnp.maximum(a0, lax.rev(b1, (0,)))
                h1 = jnp.maximum(a1, lax.rev(b0, (0,)))
                tau = jnp.minimum(jnp.min(h0), jnp.min(h1))

                lane_base = iota * _CAPL
                halfw = (nv // 2) * _L

                def pb(i, carry):
                    cx, cy = carry
                    x = blk_v[r, pl.ds(i * _L, _L)]
                    y = blk_v[r, pl.ds(halfw + i * _L, _L)]
                    mx = x >= tau
                    my = y >= tau
                    ixv = iota + i * _L
                    iyv = iota + (halfw + i * _L)
                    plsc.store_scatter(cand_v, [lane_base + cx], ixv, mask=mx)
                    plsc.store_scatter(
                        cand_v, [(lane_base + _CAPL // 2) + cy], iyv, mask=my)
                    return (cx + mx.astype(jnp.int32),
                            cy + my.astype(jnp.int32))

                zoff = jnp.zeros((_L,), jnp.int32)
                cx, cy = plsc.parallel_loop(
                    0, nv // 2, unroll=4, carry=(zoff, zoff))(pb)

                def mk_pc(cnt, boff):
                    def pc(j, st):
                        a0k, a0v, a1k, a1v = st
                        ci = plsc.load_gather(cand_v, [(lane_base + boff) + j])
                        valid = cnt > j
                        ci = jnp.where(valid, ci, 0)
                        cv = plsc.load_gather(
                            blk_v, [jnp.full((_L,), r, jnp.int32), ci])
                        cv = jnp.where(valid, cv, neginf)
                        sk, sv = plsc.sort_key_val(cv, ci, descending=True)
                        rbk = lax.rev(sk, (0,))
                        rbv = lax.rev(sv, (0,))
                        m = a1k >= rbk
                        h1k = jnp.where(m, a1k, rbk)
                        h1v = jnp.where(m, a1v, rbv)
                        m2_ = a0k >= h1k
                        uk = jnp.where(m2_, a0k, h1k)
                        uv = jnp.where(m2_, a0v, h1v)
                        lk = jnp.where(m2_, h1k, a0k)
                        lv = jnp.where(m2_, h1v, a0v)
                        a0k, a0v = plsc.sort_key_val(uk, uv, descending=True)
                        a1k, a1v = plsc.sort_key_val(lk, lv, descending=True)
                        return (a0k, a0v, a1k, a1v)
                    return pc

                zi = jnp.zeros((_L,), jnp.int32)
                st = (finf, zi, finf, zi)
                st = plsc.parallel_loop(
                    0, jnp.max(cx), carry=st)(mk_pc(cx, 0))
                st = plsc.parallel_loop(
                    0, jnp.max(cy), carry=st)(mk_pc(cy, _CAPL // 2))
                a0k, a0v, a1k, a1v = st

                ro = (blk * _RB + r) * _K
                half = iota >> 1
                plsc.store_scatter(out_v, [ro + half], a0v, mask=evenmask)
                plsc.store_scatter(
                    out_v, [(ro + _K // 2) + half], a1v, mask=evenmask)

        pltpu.async_copy(src(0), blk_a, sem_a)

        def outer(k2, carry):
            b0 = 2 * k2
            pltpu.async_copy(src(b0 + 1), blk_b, sem_b)
            pltpu.make_async_copy(src(b0), blk_a, sem_a).wait()
            process_rows(blk_a, b0)

            @pl.when(b0 + 2 < nblk)
            def _():
                pltpu.async_copy(src(b0 + 2), blk_a, sem_a)

            pltpu.make_async_copy(src(b0 + 1), blk_b, sem_b).wait()
            process_rows(blk_b, b0 + 1)
            return carry

        lax.fori_loop(0, nblk // 2, outer, jnp.int32(0))
        pltpu.sync_copy(
            out_v.at[pl.ds(0, out_words)],
            out_hbm.at[pl.ds(row_base * _K, out_words)])

    return sc_topk


def kernel(x):
    a = jnp.squeeze(x, axis=-1)
    B, D, N = a.shape
    bc = B // 4
    sc_topk = _make_sc_topk(bc * N, N)
    outs = []
    for s in range(4):
        neg2d = _neg_adj(a[:, :, :][s * bc:(s + 1) * bc])
        outs.append(sc_topk(neg2d))
    nn16 = jnp.concatenate(outs)
    nn_idx = nn16.reshape(B, N, _K)
    center = jnp.broadcast_to(
        jnp.arange(N, dtype=jnp.int32)[None, :, None], (B, N, _K))
    return jnp.stack((nn_idx, center), axis=0)


# speedup vs baseline: 1.0713x; 1.0713x over previous
import functools

import jax
import jax.numpy as jnp
from jax import lax
from jax.experimental import pallas as pl
from jax.experimental.pallas import tpu as pltpu
from jax.experimental.pallas import tpu_sc as plsc

_K = 16
_DILATION = 2
_KD = _K * _DILATION

_L = 16
_NW = 32
_RB = 8
_CAND_CAP = 256
_CAPL = 256


def _dist_kernel(a_rows_ref, a_full_ref, out_ref):
    a_rows = a_rows_ref[0]
    a_full = a_full_ref[0]
    inner = -2.0 * jax.lax.dot_general(
        a_rows, a_full, (((0,), (0,)), ((), ())),
        preferred_element_type=jnp.float32)
    sq_rows = jnp.sum(a_rows * a_rows, axis=0)
    sq_full = jnp.sum(a_full * a_full, axis=0)
    t = (sq_rows[:, None] + inner) + sq_full[None, :]
    out_ref[...] = -t


def _neg_adj(a):
    B, D, N = a.shape
    R = 256
    return pl.pallas_call(
        _dist_kernel,
        grid=(B, N // R),
        in_specs=[
            pl.BlockSpec((1, D, R), lambda b, r: (b, 0, r)),
            pl.BlockSpec((1, D, N), lambda b, r: (b, 0, 0)),
        ],
        out_specs=pl.BlockSpec((R, N), lambda b, r: (b * (N // R) + r, 0)),
        out_shape=jax.ShapeDtypeStruct((B * N, N), jnp.float32),
    )(a, a)


def _merge16(ak, bk):
    rb = lax.rev(bk, (0,))
    u = jnp.maximum(ak, rb)
    l = jnp.minimum(ak, rb)
    us, _ = plsc.sort_key_val(u, u, descending=True)
    ls, _ = plsc.sort_key_val(l, l, descending=True)
    return us, ls


def _make_sc_topk(rows_total, n):
    rows_per_w = rows_total // _NW
    nblk = rows_per_w // _RB
    nv = n // _L
    blk_words = _RB * n
    out_words = rows_per_w * _K

    mesh = plsc.VectorSubcoreMesh(core_axis_name="c", subcore_axis_name="s")

    @functools.partial(
        pl.kernel,
        mesh=mesh,
        out_type=jax.ShapeDtypeStruct((rows_total * _K,), jnp.int32),
        scratch_types=[
            pltpu.VMEM((_RB, n), jnp.float32),
            pltpu.VMEM((_RB, n), jnp.float32),
            pltpu.VMEM((_L * _CAPL,), jnp.int32),
            pltpu.VMEM((out_words + _L,), jnp.int32),
            pltpu.SemaphoreType.DMA,
            pltpu.SemaphoreType.DMA,
        ],
        compiler_params=pltpu.CompilerParams(needs_layout_passes=False),
    )
    def sc_topk(neg_hbm, out_hbm, blk_a, blk_b, cand_v, out_v, sem_a, sem_b):
        cid = lax.axis_index("c")
        sid = lax.axis_index("s")
        wid = sid * 2 + cid
        row_base = wid * rows_per_w

        neginf = jnp.float32(-jnp.inf)
        iota = lax.iota(jnp.int32, _L)
        evenmask = (iota & 1) == 0

        def src(blk):
            return neg_hbm.at[pl.ds(row_base + blk * _RB, _RB)]

        def process_rows(blk_v, blk):
            for r in range(_RB):

                def pa(i, ms):
                    m1, m2, m3, m4 = ms
                    gb = i * (4 * _L)
                    x0 = blk_v[r, pl.ds(gb, _L)]
                    x1 = blk_v[r, pl.ds(gb + _L, _L)]
                    x2 = blk_v[r, pl.ds(gb + 2 * _L, _L)]
                    x3 = blk_v[r, pl.ds(gb + 3 * _L, _L)]
                    a = jnp.maximum(x0, x1)
                    b = jnp.minimum(x0, x1)
                    c = jnp.maximum(x2, x3)
                    d = jnp.minimum(x2, x3)
                    g1 = jnp.maximum(a, c)
                    c2 = jnp.minimum(a, c)
                    b2 = jnp.maximum(b, d)
                    g4 = jnp.minimum(b, d)
                    g2 = jnp.maximum(c2, b2)
                    g3 = jnp.minimum(c2, b2)
                    h1 = jnp.maximum(m1, g4)
                    h2 = jnp.maximum(m2, g3)
                    h3 = jnp.maximum(m3, g2)
                    h4 = jnp.maximum(m4, g1)
                    p1 = jnp.maximum(h1, h3)
                    p3 = jnp.minimum(h1, h3)
                    p2 = jnp.maximum(h2, h4)
                    p4 = jnp.minimum(h2, h4)
                    return (jnp.maximum(p1, p2), jnp.minimum(p1, p2),
                            jnp.maximum(p3, p4), jnp.minimum(p3, p4))

                finf = jnp.full((_L,), neginf)
                m1, m2, m3, m4 = plsc.parallel_loop(
                    0, nv // 4, unroll=2,
                    carry=(finf, finf, finf, finf))(pa)

                s1, _ = plsc.sort_key_val(m1, m1, descending=True)
                s2, _ = plsc.sort_key_val(m2, m2, descending=True)
                s3, _ = plsc.sort_key_val(m3, m3, descending=True)
                s4, _ = plsc.sort_key_val(m4, m4, descending=True)
                a0, a1 = _merge16(s1, s2)
                b0, b1 = _merge16(s3, s4)
                h0 = jnp.maximum(a0, lax.rev(b1, (0,)))
                h1 = jnp.maximum(a1, lax.rev(b0, (0,)))
                tau = jnp.minimum(jnp.min(h0), jnp.min(h1))

                lane_base = iota * _CAPL
                halfw = (nv // 2) * _L

                def pb(i, carry):
                    cx, cy = carry
                    x = blk_v[r, pl.ds(i * _L, _L)]
                    y = blk_v[r, pl.ds(halfw + i * _L, _L)]
                    mx = x >= tau
                    my = y >= tau
                    ixv = iota + i * _L
                    iyv = iota + (halfw + i * _L)
                    plsc.store_scatter(cand_v, [lane_base + cx], ixv, mask=mx)
                    plsc.store_scatter(
                        cand_v, [(lane_base + _CAPL // 2) + cy], iyv, mask=my)
                    return (cx + mx.astype(jnp.int32),
                            cy + my.astype(jnp.int32))

                zoff = jnp.zeros((_L,), jnp.int32)
                cx, cy = plsc.parallel_loop(
                    0, nv // 2, unroll=4, carry=(zoff, zoff))(pb)

                def mk_pc(cnt, boff):
                    def pc(j, st):
                        a0k, a0v, a1k, a1v = st
                        ci = plsc.load_gather(cand_v, [(lane_base + boff) + j])
                        valid = cnt > j
                        ci = jnp.where(valid, ci, 0)
                        cv = plsc.load_gather(
                            blk_v, [jnp.full((_L,), r, jnp.int32), ci])
                        cv = jnp.where(valid, cv, neginf)
                        sk, sv = plsc.sort_key_val(cv, ci, descending=True)
                        rbk = lax.rev(sk, (0,))
                        rbv = lax.rev(sv, (0,))
                        m = a1k >= rbk
                        h1k = jnp.where(m, a1k, rbk)
                        h1v = jnp.where(m, a1v, rbv)
                        m2_ = a0k >= h1k
                        uk = jnp.where(m2_, a0k, h1k)
                        uv = jnp.where(m2_, a0v, h1v)
                        lk = jnp.where(m2_, h1k, a0k)
                        lv = jnp.where(m2_, h1v, a0v)
                        a0k, a0v = plsc.sort_key_val(uk, uv, descending=True)
                        a1k, a1v = plsc.sort_key_val(lk, lv, descending=True)
                        return (a0k, a0v, a1k, a1v)
                    return pc

                zi = jnp.zeros((_L,), jnp.int32)
                st = (finf, zi, finf, zi)
                st = plsc.parallel_loop(
                    0, jnp.max(cx), carry=st)(mk_pc(cx, 0))
                st = plsc.parallel_loop(
                    0, jnp.max(cy), carry=st)(mk_pc(cy, _CAPL // 2))
                a0k, a0v, a1k, a1v = st

                ro = (blk * _RB + r) * _K
                half = iota >> 1
                plsc.store_scatter(out_v, [ro + half], a0v, mask=evenmask)
                plsc.store_scatter(
                    out_v, [(ro + _K // 2) + half], a1v, mask=evenmask)

        pltpu.async_copy(src(0), blk_a, sem_a)

        def outer(k2, carry):
            b0 = 2 * k2
            pltpu.async_copy(src(b0 + 1), blk_b, sem_b)
            pltpu.make_async_copy(src(b0), blk_a, sem_a).wait()
            process_rows(blk_a, b0)

            @pl.when(b0 + 2 < nblk)
            def _():
                pltpu.async_copy(src(b0 + 2), blk_a, sem_a)

            pltpu.make_async_copy(src(b0 + 1), blk_b, sem_b).wait()
            process_rows(blk_b, b0 + 1)
            return carry

        lax.fori_loop(0, nblk // 2, outer, jnp.int32(0))
        pltpu.sync_copy(
            out_v.at[pl.ds(0, out_words)],
            out_hbm.at[pl.ds(row_base * _K, out_words)])

    return sc_topk


def kernel(x):
    a = jnp.squeeze(x, axis=-1)
    B, D, N = a.shape
    bc = B // 4
    sc_topk = _make_sc_topk(bc * N, N)
    outs = []
    for s in range(4):
        neg2d = _neg_adj(a[:, :, :][s * bc:(s + 1) * bc])
        outs.append(sc_topk(neg2d))
    nn16 = jnp.concatenate(outs)
    nn_idx = nn16.reshape(B, N, _K)
    center = jnp.broadcast_to(
        jnp.arange(N, dtype=jnp.int32)[None, :, None], (B, N, _K))
    return jnp.stack((nn_idx, center), axis=0)
